# baseline (speedup 1.0000x reference)
import functools

import jax
import jax.numpy as jnp
from jax import lax
from jax.experimental import pallas as pl
from jax.experimental.pallas import tpu as pltpu, tpu_sc as plsc

NC = 2
NS = 16
L = 16
NW = NC * NS
VBIT = 1 << 30


def _meta_predicate_body(T, R, BPW,
                         x_hbm, packed_hbm, r0, r1, r2, r3, r4, r5, r6, r7,
                         alpha_hbm, out_hbm,
                         xb, selb, gateb, actb, outv, a16, sem):
    rule_rows = (r0, r1, r2, r3, r4, r5, r6, r7)
    wid = lax.axis_index("s") * NC + lax.axis_index("c")
    base = wid * BPW
    nchunks = BPW // 128
    iota = lax.iota(jnp.int32, L)

    def bfly(v, op):
        for k in (1, 2, 4):
            a16[...] = v
            p = plsc.load_gather(a16, [jnp.bitwise_xor(iota, k)])
            v = op(v, p)
        return v

    pltpu.sync_copy(alpha_hbm, a16)
    av = a16[...]
    mx = bfly(av, jnp.maximum)
    ev = jnp.exp(av - mx)
    s8 = bfly(ev, jnp.add)
    wv = ev / s8
    a16[...] = wv

    pltpu.sync_copy(x_hbm.at[pl.ds(base, BPW)], xb)

    sel_d = []
    for r in range(R):
        for h in range(nchunks):
            idx = xb.at[pl.ds(h * 128, 128)]
            j = r * nchunks + h
            sel_d.append(pltpu.async_copy(
                packed_hbm.at[r].at[idx], selb.at[j], sem))

    act_d = []
    mask = jnp.full((L,), VBIT - 1, jnp.int32)
    one = jnp.full((L,), 1.0, jnp.float32)
    zero = jnp.zeros((L,), jnp.float32)
    for r in range(R):
        for h in range(nchunks):
            j = r * nchunks + h
            sel_d[j].wait()
            for c in range(128 // L):
                pv = selb[j, pl.ds(c * L, L)]
                selb[j, pl.ds(c * L, L)] = pv & mask
                gateb[j, pl.ds(c * L, L)] = jnp.where(pv < VBIT, one, zero)
            act_d.append(pltpu.async_copy(
                rule_rows[r].at[selb.at[j]], actb.at[j], sem))
    for d in act_d:
        d.wait()

    wr_splat = [plsc.load_gather(a16, [jnp.full((L,), r + 8, jnp.int32)])
                for r in range(R)]
    nm = BPW // L
    for m in range(nm):
        acc = jnp.zeros((L,), jnp.float32)
        for r in range(R):
            i0 = r * BPW + m * L
            j, c = i0 // 128, i0 % 128
            actv = actb[j, pl.ds(c, L)]
            gatev = gateb[j, pl.ds(c, L)]
            acc = acc + actv * wr_splat[r] * gatev
        outv[pl.ds(m * L, L)] = acc

    pltpu.sync_copy(outv, out_hbm.at[pl.ds(base, BPW)])


def kernel(x, mat_idx, valid_gate, rule_outputs, alpha_w):
    T, R = mat_idx.shape
    B = x.shape[0]
    BPW = B // NW
    nrows = BPW * R // 128
    mat_t = mat_idx.T
    gate_t = valid_gate.T
    packed = jnp.where(gate_t >= 0.25, mat_t, mat_t | VBIT)
    rule_rows = [rule_outputs[r, :, 0] for r in range(R)]
    alpha16 = jnp.tile(alpha_w, 2)
    x = x.astype(jnp.int32)

    mesh = plsc.VectorSubcoreMesh(core_axis_name="c", subcore_axis_name="s",
                                  num_cores=NC, num_subcores=NS)
    body = functools.partial(_meta_predicate_body, T, R, BPW)
    run = pl.kernel(
        body,
        out_type=jax.ShapeDtypeStruct((B,), jnp.float32),
        mesh=mesh,
        compiler_params=pltpu.CompilerParams(needs_layout_passes=False,
                                             use_tc_tiling_on_sc=False),
        scratch_types=[
            pltpu.VMEM((BPW,), jnp.int32),
            pltpu.VMEM((nrows, 128), jnp.int32),
            pltpu.VMEM((nrows, 128), jnp.float32),
            pltpu.VMEM((nrows, 128), jnp.float32),
            pltpu.VMEM((BPW,), jnp.float32),
            pltpu.VMEM((16,), jnp.float32),
            pltpu.SemaphoreType.DMA,
        ],
    )
    ret = run(x, packed, *rule_rows, alpha16)
    return (ret.reshape(B, 1), jnp.zeros(()))


# speedup vs baseline: 1.0280x; 1.0280x over previous
import functools

import jax
import jax.numpy as jnp
from jax import lax
from jax.experimental import pallas as pl
from jax.experimental.pallas import tpu as pltpu, tpu_sc as plsc

NC = 2
NS = 16
L = 16
NW = NC * NS
VBIT = 1 << 30


def _meta_predicate_body(T, R, BPW,
                         x_hbm, packed_hbm, rule_hbm, alpha_hbm,
                         out_hbm,
                         xb, selb, gateb, actb, outv, a16, sem):
    wid = lax.axis_index("s") * NC + lax.axis_index("c")
    base = wid * BPW
    nchunks = BPW // 128
    iota = lax.iota(jnp.int32, L)

    def bfly(v, op):
        for k in (1, 2, 4):
            a16[...] = v
            p = plsc.load_gather(a16, [jnp.bitwise_xor(iota, k)])
            v = op(v, p)
        return v

    pltpu.sync_copy(alpha_hbm, a16)
    av = a16[...]
    mx = bfly(av, jnp.maximum)
    ev = jnp.exp(av - mx)
    s8 = bfly(ev, jnp.add)
    wv = ev / s8
    a16[...] = wv

    pltpu.sync_copy(x_hbm.at[pl.ds(base, BPW)], xb)

    sel_d = [pltpu.async_copy(packed_hbm.at[r].at[xb], selb.at[r], sem)
             for r in range(R)]

    act_d = []
    mask = jnp.full((L,), VBIT - 1, jnp.int32)
    one = jnp.full((L,), 1.0, jnp.float32)
    zero = jnp.zeros((L,), jnp.float32)
    nm = BPW // L
    for r in range(R):
        sel_d[r].wait()
        for c in range(nm):
            pv = selb[r, pl.ds(c * L, L)]
            selb[r, pl.ds(c * L, L)] = pv & mask
            gateb[r, pl.ds(c * L, L)] = jnp.where(pv < VBIT, one, zero)
        act_d.append(pltpu.async_copy(
            rule_hbm.at[r].at[selb.at[r]], actb.at[r], sem))
    for d in act_d:
        d.wait()

    wr_splat = [plsc.load_gather(a16, [jnp.full((L,), r + 8, jnp.int32)])
                for r in range(R)]
    for m in range(nm):
        acc = jnp.zeros((L,), jnp.float32)
        for r in range(R):
            actv = actb[r, pl.ds(m * L, L)]
            gatev = gateb[r, pl.ds(m * L, L)]
            acc = acc + actv * wr_splat[r] * gatev
        outv[pl.ds(m * L, L)] = acc

    pltpu.sync_copy(outv, out_hbm.at[pl.ds(base, BPW)])


def kernel(x, mat_idx, valid_gate, rule_outputs, alpha_w):
    T, R = mat_idx.shape
    B = x.shape[0]
    BPW = B // NW
    nrows = BPW * R // 128
    mat_t = mat_idx.T
    gate_t = valid_gate.T
    packed = jnp.where(gate_t >= 0.25, mat_t, mat_t | VBIT)
    rule_t = rule_outputs.reshape(R, T)
    alpha16 = jnp.tile(alpha_w, 2)
    x = x.astype(jnp.int32)

    mesh = plsc.VectorSubcoreMesh(core_axis_name="c", subcore_axis_name="s",
                                  num_cores=NC, num_subcores=NS)
    body = functools.partial(_meta_predicate_body, T, R, BPW)
    run = pl.kernel(
        body,
        out_type=jax.ShapeDtypeStruct((B,), jnp.float32),
        mesh=mesh,
        compiler_params=pltpu.CompilerParams(needs_layout_passes=False,
                                             use_tc_tiling_on_sc=False),
        scratch_types=[
            pltpu.VMEM((BPW,), jnp.int32),
            pltpu.VMEM((R, BPW), jnp.int32),
            pltpu.VMEM((R, BPW), jnp.float32),
            pltpu.VMEM((R, BPW), jnp.float32),
            pltpu.VMEM((BPW,), jnp.float32),
            pltpu.VMEM((16,), jnp.float32),
            pltpu.SemaphoreType.DMA,
        ],
    )
    ret = run(x, packed, rule_t, alpha16)
    return (ret.reshape(B, 1), jnp.zeros(()))


# speedup vs baseline: 1.1009x; 1.0709x over previous
import functools

import jax
import jax.numpy as jnp
from jax import lax
from jax.experimental import pallas as pl
from jax.experimental.pallas import tpu as pltpu, tpu_sc as plsc

NC = 2
NS = 16
L = 16
NW = NC * NS
VBIT = 1 << 30


def _gather_body(T, R, BPW,
                 x_hbm, packed_hbm, sel_hbm,
                 xb, selb, sem):
    wid = lax.axis_index("s") * NC + lax.axis_index("c")
    base = wid * BPW

    pltpu.sync_copy(x_hbm.at[pl.ds(base, BPW)], xb)
    sel_d = [pltpu.async_copy(packed_hbm.at[r].at[xb], selb.at[r], sem)
             for r in range(R)]
    for d in sel_d:
        d.wait()
    pltpu.sync_copy(selb, sel_hbm.at[wid])


def _combine_body(T, R, BPW,
                  sel_hbm, rule_hbm, alpha_hbm, out_hbm,
                  selb, gateb, actb, outv, a16, sem):
    wid = lax.axis_index("s") * NC + lax.axis_index("c")
    base = wid * BPW
    iota = lax.iota(jnp.int32, L)

    def bfly(v, op):
        for k in (1, 2, 4):
            a16[...] = v
            p = plsc.load_gather(a16, [jnp.bitwise_xor(iota, k)])
            v = op(v, p)
        return v

    pltpu.sync_copy(alpha_hbm, a16)
    av = a16[...]
    mx = bfly(av, jnp.maximum)
    ev = jnp.exp(av - mx)
    s8 = bfly(ev, jnp.add)
    wv = ev / s8
    a16[...] = wv

    pltpu.sync_copy(sel_hbm.at[wid], selb)

    act_d = []
    mask = jnp.full((L,), VBIT - 1, jnp.int32)
    one = jnp.full((L,), 1.0, jnp.float32)
    zero = jnp.zeros((L,), jnp.float32)
    nm = BPW // L
    for r in range(R):
        for c in range(nm):
            pv = selb[r, pl.ds(c * L, L)]
            selb[r, pl.ds(c * L, L)] = pv & mask
            gateb[r, pl.ds(c * L, L)] = jnp.where(pv < VBIT, one, zero)
        act_d.append(pltpu.async_copy(
            rule_hbm.at[r].at[selb.at[r]], actb.at[r], sem))
    for d in act_d:
        d.wait()

    wr_splat = [plsc.load_gather(a16, [jnp.full((L,), r + 8, jnp.int32)])
                for r in range(R)]
    for m in range(nm):
        acc = jnp.zeros((L,), jnp.float32)
        for r in range(R):
            actv = actb[r, pl.ds(m * L, L)]
            gatev = gateb[r, pl.ds(m * L, L)]
            acc = acc + actv * wr_splat[r] * gatev
        outv[pl.ds(m * L, L)] = acc

    pltpu.sync_copy(outv, out_hbm.at[pl.ds(base, BPW)])


def kernel(x, mat_idx, valid_gate, rule_outputs, alpha_w):
    T, R = mat_idx.shape
    B = x.shape[0]
    BPW = B // NW
    mat_t = mat_idx.T
    gate_t = valid_gate.T
    packed = jnp.where(gate_t >= 0.25, mat_t, mat_t | VBIT)
    rule_t = rule_outputs.reshape(R, T)
    alpha16 = jnp.tile(alpha_w, 2)
    x = x.astype(jnp.int32)

    mesh = plsc.VectorSubcoreMesh(core_axis_name="c", subcore_axis_name="s",
                                  num_cores=NC, num_subcores=NS)
    cp = pltpu.CompilerParams(needs_layout_passes=False,
                              use_tc_tiling_on_sc=False)

    gather = pl.kernel(
        functools.partial(_gather_body, T, R, BPW),
        out_type=jax.ShapeDtypeStruct((NW, R, BPW), jnp.int32),
        mesh=mesh,
        compiler_params=cp,
        scratch_types=[
            pltpu.VMEM((BPW,), jnp.int32),
            pltpu.VMEM((R, BPW), jnp.int32),
            pltpu.SemaphoreType.DMA,
        ],
    )
    combine = pl.kernel(
        functools.partial(_combine_body, T, R, BPW),
        out_type=jax.ShapeDtypeStruct((B,), jnp.float32),
        mesh=mesh,
        compiler_params=cp,
        scratch_types=[
            pltpu.VMEM((R, BPW), jnp.int32),
            pltpu.VMEM((R, BPW), jnp.float32),
            pltpu.VMEM((R, BPW), jnp.float32),
            pltpu.VMEM((BPW,), jnp.float32),
            pltpu.VMEM((16,), jnp.float32),
            pltpu.SemaphoreType.DMA,
        ],
    )
    sel = gather(x, packed)
    ret = combine(sel, rule_t, alpha16)
    return (ret.reshape(B, 1), jnp.zeros(()))
